# Initial kernel scaffold; baseline (speedup 1.0000x reference)
import jax
import jax.numpy as jnp
from jax.experimental import pallas as pl


def kernel(features, edge_index, W1, b1, W2, b2):
    raise NotImplementedError("write your pallas kernel here")


# speedup vs baseline: 7.8697x; 7.8697x over previous
import functools

import jax
import jax.numpy as jnp
from jax import lax
from jax.experimental import pallas as pl
from jax.experimental.pallas import tpu as pltpu
from jax.experimental.pallas import tpu_sc as plsc

N = 10000
E = 320000
D = 128

NC = 2
NS = 16

B = 125
ROWS_PER_TILE = E // (NC * NS) // B
EDGE_ROWS = E // B
N_PAD = 10240
NPT = N_PAD // NS
ZCH = 80
ZROWS = NPT // ZCH

_MESH = plsc.VectorSubcoreMesh(core_axis_name="c", subcore_axis_name="s")


def _zero_vmem(ref, nrows, ncols):
    zeros16 = jnp.zeros((16,), jnp.float32)

    def body(i, _):
        for col in range(ncols // 16):
            ref[i, pl.ds(col * 16, 16)] = zeros16
        return 0

    lax.fori_loop(0, nrows, body, 0)


def _deg_body(src_hbm, dst_hbm, out_hbm,
              deg_s, deg_d, src_v, dst_v, ones_v, zbuf):
    c = lax.axis_index("c")
    s = lax.axis_index("s")

    _zero_vmem(zbuf, ZROWS, 16)
    for k in range(ZCH):
        base = s * NPT + k * ZROWS
        pltpu.sync_copy(zbuf, deg_s.at[pl.ds(base, ZROWS)])
        pltpu.sync_copy(zbuf, deg_d.at[pl.ds(base, ZROWS)])

    ones16 = jnp.ones((16,), jnp.float32)

    def ones_body(i, _):
        ones_v[i, :] = ones16
        return 0

    lax.fori_loop(0, B, ones_body, 0)

    plsc.subcore_barrier()

    row0 = c * (EDGE_ROWS // NC) + s * ROWS_PER_TILE
    pltpu.sync_copy(src_hbm.at[pl.ds(row0, ROWS_PER_TILE)], src_v)
    pltpu.sync_copy(dst_hbm.at[pl.ds(row0, ROWS_PER_TILE)], dst_v)

    def chunk(j, _):
        pltpu.sync_copy(ones_v, deg_s.at[src_v.at[j]], add=True)
        pltpu.sync_copy(ones_v, deg_d.at[dst_v.at[j]], add=True)
        return 0

    lax.fori_loop(0, ROWS_PER_TILE, chunk, 0)

    plsc.subcore_barrier()

    base = s * NPT
    pltpu.sync_copy(deg_s.at[pl.ds(base, NPT)], out_hbm.at[c, 0, pl.ds(base, NPT)])
    pltpu.sync_copy(deg_d.at[pl.ds(base, NPT)], out_hbm.at[c, 1, pl.ds(base, NPT)])


_SC_PARAMS = pltpu.CompilerParams(use_tc_tiling_on_sc=False)

_deg_kernel = pl.kernel(
    _deg_body,
    out_type=jax.ShapeDtypeStruct((NC, 2, N_PAD, 16), jnp.float32),
    mesh=_MESH,
    compiler_params=_SC_PARAMS,
    scratch_types=[
        pltpu.VMEM_SHARED((N_PAD, 16), jnp.float32),
        pltpu.VMEM_SHARED((N_PAD, 16), jnp.float32),
        pltpu.VMEM((ROWS_PER_TILE, B), jnp.int32),
        pltpu.VMEM((ROWS_PER_TILE, B), jnp.int32),
        pltpu.VMEM((B, 16), jnp.float32),
        pltpu.VMEM((ZROWS, 16), jnp.float32),
    ],
)


def _agg_body(h_hbm, src_hbm, dst_hbm, out_hbm,
              acc, src_v, dst_v, rows_v, zbuf, sem):
    c = lax.axis_index("c")
    s = lax.axis_index("s")

    _zero_vmem(zbuf, ZROWS, D)
    for k in range(ZCH):
        base = s * NPT + k * ZROWS
        pltpu.sync_copy(zbuf, acc.at[pl.ds(base, ZROWS)])

    plsc.subcore_barrier()

    row0 = c * (EDGE_ROWS // NC) + s * ROWS_PER_TILE
    pltpu.sync_copy(src_hbm.at[pl.ds(row0, ROWS_PER_TILE)], src_v)
    pltpu.sync_copy(dst_hbm.at[pl.ds(row0, ROWS_PER_TILE)], dst_v)

    def chunk(j, _):
        pltpu.async_copy(h_hbm.at[src_v.at[j]], rows_v, sem).wait()
        pltpu.sync_copy(rows_v, acc.at[dst_v.at[j]], add=True)
        return 0

    lax.fori_loop(0, ROWS_PER_TILE, chunk, 0)

    plsc.subcore_barrier()

    base = s * NPT
    pltpu.sync_copy(acc.at[pl.ds(base, NPT)], out_hbm.at[c, pl.ds(base, NPT)])


_agg_kernel = pl.kernel(
    _agg_body,
    out_type=jax.ShapeDtypeStruct((NC, N_PAD, D), jnp.float32),
    mesh=_MESH,
    compiler_params=_SC_PARAMS,
    scratch_types=[
        pltpu.VMEM_SHARED((N_PAD, D), jnp.float32),
        pltpu.VMEM((ROWS_PER_TILE, B), jnp.int32),
        pltpu.VMEM((ROWS_PER_TILE, B), jnp.int32),
        pltpu.VMEM((B, D), jnp.float32),
        pltpu.VMEM((ZROWS, D), jnp.float32),
        pltpu.SemaphoreType.DMA,
    ],
)


RB = 400
GRID = N // RB


def _norms(deg_blk):
    d_src = deg_blk[0, 0] + deg_blk[1, 0]
    d_dst = deg_blk[0, 1] + deg_blk[1, 1]
    n_out = lax.rsqrt(jnp.clip(d_src[:, 0:1], 1.0, None))
    n_in = lax.rsqrt(jnp.clip(d_dst[:, 0:1], 1.0, None))
    return n_out, n_in


def _mm1_body(x_ref, w_ref, deg_ref, o_ref):
    n_out, _ = _norms(deg_ref[...])
    o_ref[...] = jnp.dot(x_ref[...], w_ref[...],
                         preferred_element_type=jnp.float32) * n_out


def _layer2_body(p_ref, deg_ref, b1_ref, w2_ref, o_ref):
    n_out, n_in = _norms(deg_ref[...])
    h = jnp.maximum((p_ref[0] + p_ref[1]) * n_in + b1_ref[...], 0.0)
    o_ref[...] = jnp.dot(h, w2_ref[...],
                         preferred_element_type=jnp.float32) * n_out


def _final_body(p_ref, deg_ref, b2_ref, o_ref):
    _, n_in = _norms(deg_ref[...])
    o_ref[...] = (p_ref[0] + p_ref[1]) * n_in + b2_ref[...]


_deg_spec = pl.BlockSpec((2, 2, RB, 16), lambda i: (0, 0, i, 0))
_row_spec = pl.BlockSpec((RB, D), lambda i: (i, 0))
_pair_spec = pl.BlockSpec((2, RB, D), lambda i: (0, i, 0))
_w_spec = pl.BlockSpec((D, D), lambda i: (0, 0))
_b_spec = pl.BlockSpec((1, D), lambda i: (0, 0))

_mm1 = pl.pallas_call(
    _mm1_body,
    grid=(GRID,),
    in_specs=[_row_spec, _w_spec, _deg_spec],
    out_specs=_row_spec,
    out_shape=jax.ShapeDtypeStruct((N, D), jnp.float32),
)

_layer2 = pl.pallas_call(
    _layer2_body,
    grid=(GRID,),
    in_specs=[_pair_spec, _deg_spec, _b_spec, _w_spec],
    out_specs=_row_spec,
    out_shape=jax.ShapeDtypeStruct((N, D), jnp.float32),
)

_final = pl.pallas_call(
    _final_body,
    grid=(GRID,),
    in_specs=[_pair_spec, _deg_spec, _b_spec],
    out_specs=_row_spec,
    out_shape=jax.ShapeDtypeStruct((N, D), jnp.float32),
)


def kernel(features, edge_index, W1, b1, W2, b2):
    src = edge_index[0].reshape(EDGE_ROWS, B)
    dst = edge_index[1].reshape(EDGE_ROWS, B)
    b1r = b1.reshape(1, D)
    b2r = b2.reshape(1, D)

    degs = _deg_kernel(src, dst)

    h1 = _mm1(features, W1, degs)
    p1 = _agg_kernel(h1, src, dst)
    h2 = _layer2(p1, degs, b1r, W2)
    p2 = _agg_kernel(h2, src, dst)
    return _final(p2, degs, b2r)


# speedup vs baseline: 10.1527x; 1.2901x over previous
import functools

import jax
import jax.numpy as jnp
from jax import lax
from jax.experimental import pallas as pl
from jax.experimental.pallas import tpu as pltpu
from jax.experimental.pallas import tpu_sc as plsc

N = 10000
E = 320000
D = 128

NC = 2
NS = 16

B = 80
ROWS_PER_TILE = E // (NC * NS) // B
EDGE_ROWS = E // B
N_PAD = 10240
NPT = N_PAD // NS
ZCH = 80
ZROWS = NPT // ZCH

_MESH = plsc.VectorSubcoreMesh(core_axis_name="c", subcore_axis_name="s")


def _zero_vmem(ref, nrows, ncols):
    zeros16 = jnp.zeros((16,), jnp.float32)

    def body(i, _):
        for col in range(ncols // 16):
            ref[i, pl.ds(col * 16, 16)] = zeros16
        return 0

    lax.fori_loop(0, nrows, body, 0)


def _deg_body(src_hbm, dst_hbm, out_hbm,
              deg_s, deg_d, src_v, dst_v, ones_v, zbuf):
    c = lax.axis_index("c")
    s = lax.axis_index("s")

    _zero_vmem(zbuf, ZROWS, 16)
    for k in range(ZCH):
        base = s * NPT + k * ZROWS
        pltpu.sync_copy(zbuf, deg_s.at[pl.ds(base, ZROWS)])
        pltpu.sync_copy(zbuf, deg_d.at[pl.ds(base, ZROWS)])

    ones16 = jnp.ones((16,), jnp.float32)

    def ones_body(i, _):
        ones_v[i, :] = ones16
        return 0

    lax.fori_loop(0, B, ones_body, 0)

    plsc.subcore_barrier()

    row0 = c * (EDGE_ROWS // NC) + s * ROWS_PER_TILE
    pltpu.sync_copy(src_hbm.at[pl.ds(row0, ROWS_PER_TILE)], src_v)
    pltpu.sync_copy(dst_hbm.at[pl.ds(row0, ROWS_PER_TILE)], dst_v)

    def chunk(j, _):
        pltpu.sync_copy(ones_v, deg_s.at[src_v.at[j]], add=True)
        pltpu.sync_copy(ones_v, deg_d.at[dst_v.at[j]], add=True)
        return 0

    lax.fori_loop(0, ROWS_PER_TILE, chunk, 0)

    plsc.subcore_barrier()

    base = s * NPT
    pltpu.sync_copy(deg_s.at[pl.ds(base, NPT)], out_hbm.at[c, 0, pl.ds(base, NPT)])
    pltpu.sync_copy(deg_d.at[pl.ds(base, NPT)], out_hbm.at[c, 1, pl.ds(base, NPT)])


_SC_PARAMS = pltpu.CompilerParams(use_tc_tiling_on_sc=False)

_deg_kernel = pl.kernel(
    _deg_body,
    out_type=jax.ShapeDtypeStruct((NC, 2, N_PAD, 16), jnp.float32),
    mesh=_MESH,
    compiler_params=_SC_PARAMS,
    scratch_types=[
        pltpu.VMEM_SHARED((N_PAD, 16), jnp.float32),
        pltpu.VMEM_SHARED((N_PAD, 16), jnp.float32),
        pltpu.VMEM((ROWS_PER_TILE, B), jnp.int32),
        pltpu.VMEM((ROWS_PER_TILE, B), jnp.int32),
        pltpu.VMEM((B, 16), jnp.float32),
        pltpu.VMEM((ZROWS, 16), jnp.float32),
    ],
)


def _agg_body(h_hbm, src_hbm, dst_hbm, out_hbm,
              acc, src_v, dst_v, rows0, rows1, zbuf, sem0, sem1):
    c = lax.axis_index("c")
    s = lax.axis_index("s")

    _zero_vmem(zbuf, ZROWS, D)
    for k in range(ZCH):
        base = s * NPT + k * ZROWS
        pltpu.sync_copy(zbuf, acc.at[pl.ds(base, ZROWS)])

    plsc.subcore_barrier()

    row0 = c * (EDGE_ROWS // NC) + s * ROWS_PER_TILE
    pltpu.sync_copy(src_hbm.at[pl.ds(row0, ROWS_PER_TILE)], src_v)
    pltpu.sync_copy(dst_hbm.at[pl.ds(row0, ROWS_PER_TILE)], dst_v)

    nch = ROWS_PER_TILE
    pltpu.async_copy(h_hbm.at[src_v.at[0]], rows0, sem0)

    def chunk2(j2, _):
        j = j2 * 2
        pltpu.async_copy(h_hbm.at[src_v.at[j + 1]], rows1, sem1)
        pltpu.make_async_copy(h_hbm.at[src_v.at[j]], rows0, sem0).wait()
        pltpu.sync_copy(rows0, acc.at[dst_v.at[j]], add=True)

        @pl.when(j + 2 < nch)
        def _():
            pltpu.async_copy(h_hbm.at[src_v.at[j + 2]], rows0, sem0)

        pltpu.make_async_copy(h_hbm.at[src_v.at[j + 1]], rows1, sem1).wait()
        pltpu.sync_copy(rows1, acc.at[dst_v.at[j + 1]], add=True)
        return 0

    lax.fori_loop(0, nch // 2, chunk2, 0)

    pltpu.make_async_copy(h_hbm.at[src_v.at[nch - 1]], rows0, sem0).wait()
    pltpu.sync_copy(rows0, acc.at[dst_v.at[nch - 1]], add=True)

    plsc.subcore_barrier()

    base = s * NPT
    pltpu.sync_copy(acc.at[pl.ds(base, NPT)], out_hbm.at[c, pl.ds(base, NPT)])


_agg_kernel = pl.kernel(
    _agg_body,
    out_type=jax.ShapeDtypeStruct((NC, N_PAD, D), jnp.float32),
    mesh=_MESH,
    compiler_params=_SC_PARAMS,
    scratch_types=[
        pltpu.VMEM_SHARED((N_PAD, D), jnp.float32),
        pltpu.VMEM((ROWS_PER_TILE, B), jnp.int32),
        pltpu.VMEM((ROWS_PER_TILE, B), jnp.int32),
        pltpu.VMEM((B, D), jnp.float32),
        pltpu.VMEM((B, D), jnp.float32),
        pltpu.VMEM((ZROWS, D), jnp.float32),
        pltpu.SemaphoreType.DMA,
        pltpu.SemaphoreType.DMA,
    ],
)


RB = 400
GRID = N // RB


def _norms(deg_blk):
    d_src = deg_blk[0, 0] + deg_blk[1, 0]
    d_dst = deg_blk[0, 1] + deg_blk[1, 1]
    n_out = lax.rsqrt(jnp.clip(d_src[:, 0:1], 1.0, None))
    n_in = lax.rsqrt(jnp.clip(d_dst[:, 0:1], 1.0, None))
    return n_out, n_in


def _mm1_body(x_ref, w_ref, deg_ref, o_ref):
    n_out, _ = _norms(deg_ref[...])
    o_ref[...] = jnp.dot(x_ref[...], w_ref[...],
                         preferred_element_type=jnp.float32) * n_out


def _layer2_body(p_ref, deg_ref, b1_ref, w2_ref, o_ref):
    n_out, n_in = _norms(deg_ref[...])
    h = jnp.maximum((p_ref[0] + p_ref[1]) * n_in + b1_ref[...], 0.0)
    o_ref[...] = jnp.dot(h, w2_ref[...],
                         preferred_element_type=jnp.float32) * n_out


def _final_body(p_ref, deg_ref, b2_ref, o_ref):
    _, n_in = _norms(deg_ref[...])
    o_ref[...] = (p_ref[0] + p_ref[1]) * n_in + b2_ref[...]


_deg_spec = pl.BlockSpec((2, 2, RB, 16), lambda i: (0, 0, i, 0))
_row_spec = pl.BlockSpec((RB, D), lambda i: (i, 0))
_pair_spec = pl.BlockSpec((2, RB, D), lambda i: (0, i, 0))
_w_spec = pl.BlockSpec((D, D), lambda i: (0, 0))
_b_spec = pl.BlockSpec((1, D), lambda i: (0, 0))

_mm1 = pl.pallas_call(
    _mm1_body,
    grid=(GRID,),
    in_specs=[_row_spec, _w_spec, _deg_spec],
    out_specs=_row_spec,
    out_shape=jax.ShapeDtypeStruct((N, D), jnp.float32),
)

_layer2 = pl.pallas_call(
    _layer2_body,
    grid=(GRID,),
    in_specs=[_pair_spec, _deg_spec, _b_spec, _w_spec],
    out_specs=_row_spec,
    out_shape=jax.ShapeDtypeStruct((N, D), jnp.float32),
)

_final = pl.pallas_call(
    _final_body,
    grid=(GRID,),
    in_specs=[_pair_spec, _deg_spec, _b_spec],
    out_specs=_row_spec,
    out_shape=jax.ShapeDtypeStruct((N, D), jnp.float32),
)


def kernel(features, edge_index, W1, b1, W2, b2):
    src = edge_index[0].reshape(EDGE_ROWS, B)
    dst = edge_index[1].reshape(EDGE_ROWS, B)
    b1r = b1.reshape(1, D)
    b2r = b2.reshape(1, D)

    degs = _deg_kernel(src, dst)

    h1 = _mm1(features, W1, degs)
    p1 = _agg_kernel(h1, src, dst)
    h2 = _layer2(p1, degs, b1r, W2)
    p2 = _agg_kernel(h2, src, dst)
    return _final(p2, degs, b2r)


# speedup vs baseline: 10.6385x; 1.0479x over previous
import functools

import jax
import jax.numpy as jnp
from jax import lax
from jax.experimental import pallas as pl
from jax.experimental.pallas import tpu as pltpu
from jax.experimental.pallas import tpu_sc as plsc

N = 10000
E = 320000
D = 128

NC = 2
NS = 16

B = 125
ROWS_PER_TILE = E // (NC * NS) // B
EDGE_ROWS = E // B
HALF = ROWS_PER_TILE // 2
N_PAD = 10240
NPT = N_PAD // NS
ZCH = 80
ZROWS = NPT // ZCH

_MESH = plsc.VectorSubcoreMesh(core_axis_name="c", subcore_axis_name="s")


def _zero_vmem(ref, nrows, ncols):
    zeros16 = jnp.zeros((16,), jnp.float32)

    def body(i, _):
        for col in range(ncols // 16):
            ref[i, pl.ds(col * 16, 16)] = zeros16
        return 0

    lax.fori_loop(0, nrows, body, 0)


def _deg_body(src_hbm, dst_hbm, out_hbm,
              deg_s, deg_d, src_v, dst_v, ones_v, zbuf):
    c = lax.axis_index("c")
    s = lax.axis_index("s")

    _zero_vmem(zbuf, ZROWS, 16)
    for k in range(ZCH):
        base = s * NPT + k * ZROWS
        pltpu.sync_copy(zbuf, deg_s.at[pl.ds(base, ZROWS)])
        pltpu.sync_copy(zbuf, deg_d.at[pl.ds(base, ZROWS)])

    ones16 = jnp.ones((16,), jnp.float32)

    def ones_body(i, _):
        ones_v[i, :] = ones16
        return 0

    lax.fori_loop(0, B, ones_body, 0)

    plsc.subcore_barrier()

    row0 = c * (EDGE_ROWS // NC) + s * ROWS_PER_TILE
    pltpu.sync_copy(src_hbm.at[pl.ds(row0, ROWS_PER_TILE)], src_v)
    pltpu.sync_copy(dst_hbm.at[pl.ds(row0, ROWS_PER_TILE)], dst_v)

    def chunk(j, _):
        pltpu.sync_copy(ones_v, deg_s.at[src_v.at[j]], add=True)
        pltpu.sync_copy(ones_v, deg_d.at[dst_v.at[j]], add=True)
        return 0

    lax.fori_loop(0, ROWS_PER_TILE, chunk, 0)

    plsc.subcore_barrier()

    base = s * NPT
    pltpu.sync_copy(deg_s.at[pl.ds(base, NPT)], out_hbm.at[c, 0, pl.ds(base, NPT)])
    pltpu.sync_copy(deg_d.at[pl.ds(base, NPT)], out_hbm.at[c, 1, pl.ds(base, NPT)])


_SC_PARAMS = pltpu.CompilerParams(use_tc_tiling_on_sc=False)

_deg_kernel = pl.kernel(
    _deg_body,
    out_type=jax.ShapeDtypeStruct((NC, 2, N_PAD, 16), jnp.float32),
    mesh=_MESH,
    compiler_params=_SC_PARAMS,
    scratch_types=[
        pltpu.VMEM_SHARED((N_PAD, 16), jnp.float32),
        pltpu.VMEM_SHARED((N_PAD, 16), jnp.float32),
        pltpu.VMEM((ROWS_PER_TILE, B), jnp.int32),
        pltpu.VMEM((ROWS_PER_TILE, B), jnp.int32),
        pltpu.VMEM((B, 16), jnp.float32),
        pltpu.VMEM((ZROWS, 16), jnp.float32),
    ],
)


def _agg_body(h_hbm, src_hbm, dst_hbm, out_hbm,
              acc, src_v, dst_v, rows0, rows1, zbuf, sem0, sem1):
    c = lax.axis_index("c")
    s = lax.axis_index("s")

    _zero_vmem(zbuf, ZROWS, D)
    for k in range(ZCH):
        base = s * NPT + k * ZROWS
        pltpu.sync_copy(zbuf, acc.at[pl.ds(base, ZROWS)])

    plsc.subcore_barrier()

    row0 = c * (EDGE_ROWS // NC) + s * ROWS_PER_TILE

    for half in range(2):
        hrow = row0 + half * HALF
        pltpu.sync_copy(src_hbm.at[pl.ds(hrow, HALF)], src_v)
        pltpu.sync_copy(dst_hbm.at[pl.ds(hrow, HALF)], dst_v)

        pltpu.async_copy(h_hbm.at[src_v.at[0]], rows0, sem0)

        def chunk2(j2, _):
            j = j2 * 2
            pltpu.async_copy(h_hbm.at[src_v.at[j + 1]], rows1, sem1)
            pltpu.make_async_copy(h_hbm.at[src_v.at[j]], rows0, sem0).wait()
            pltpu.sync_copy(rows0, acc.at[dst_v.at[j]], add=True)

            @pl.when(j + 2 < HALF)
            def _():
                pltpu.async_copy(h_hbm.at[src_v.at[j + 2]], rows0, sem0)

            pltpu.make_async_copy(h_hbm.at[src_v.at[j + 1]], rows1, sem1).wait()
            pltpu.sync_copy(rows1, acc.at[dst_v.at[j + 1]], add=True)
            return 0

        lax.fori_loop(0, HALF // 2, chunk2, 0)

    plsc.subcore_barrier()

    base = s * NPT
    pltpu.sync_copy(acc.at[pl.ds(base, NPT)], out_hbm.at[c, pl.ds(base, NPT)])


_agg_kernel = pl.kernel(
    _agg_body,
    out_type=jax.ShapeDtypeStruct((NC, N_PAD, D), jnp.float32),
    mesh=_MESH,
    compiler_params=_SC_PARAMS,
    scratch_types=[
        pltpu.VMEM_SHARED((N_PAD, D), jnp.float32),
        pltpu.VMEM((HALF, B), jnp.int32),
        pltpu.VMEM((HALF, B), jnp.int32),
        pltpu.VMEM((B, D), jnp.float32),
        pltpu.VMEM((B, D), jnp.float32),
        pltpu.VMEM((ZROWS, D), jnp.float32),
        pltpu.SemaphoreType.DMA,
        pltpu.SemaphoreType.DMA,
    ],
)


RB = 400
GRID = N // RB


def _norms(deg_blk):
    d_src = deg_blk[0, 0] + deg_blk[1, 0]
    d_dst = deg_blk[0, 1] + deg_blk[1, 1]
    n_out = lax.rsqrt(jnp.clip(d_src[:, 0:1], 1.0, None))
    n_in = lax.rsqrt(jnp.clip(d_dst[:, 0:1], 1.0, None))
    return n_out, n_in


def _mm1_body(x_ref, w_ref, deg_ref, o_ref):
    n_out, _ = _norms(deg_ref[...])
    o_ref[...] = jnp.dot(x_ref[...], w_ref[...],
                         preferred_element_type=jnp.float32) * n_out


def _layer2_body(p_ref, deg_ref, b1_ref, w2_ref, o_ref):
    n_out, n_in = _norms(deg_ref[...])
    h = jnp.maximum((p_ref[0] + p_ref[1]) * n_in + b1_ref[...], 0.0)
    o_ref[...] = jnp.dot(h, w2_ref[...],
                         preferred_element_type=jnp.float32) * n_out


def _final_body(p_ref, deg_ref, b2_ref, o_ref):
    _, n_in = _norms(deg_ref[...])
    o_ref[...] = (p_ref[0] + p_ref[1]) * n_in + b2_ref[...]


_deg_spec = pl.BlockSpec((2, 2, RB, 16), lambda i: (0, 0, i, 0))
_row_spec = pl.BlockSpec((RB, D), lambda i: (i, 0))
_pair_spec = pl.BlockSpec((2, RB, D), lambda i: (0, i, 0))
_w_spec = pl.BlockSpec((D, D), lambda i: (0, 0))
_b_spec = pl.BlockSpec((1, D), lambda i: (0, 0))

_mm1 = pl.pallas_call(
    _mm1_body,
    grid=(GRID,),
    in_specs=[_row_spec, _w_spec, _deg_spec],
    out_specs=_row_spec,
    out_shape=jax.ShapeDtypeStruct((N, D), jnp.float32),
)

_layer2 = pl.pallas_call(
    _layer2_body,
    grid=(GRID,),
    in_specs=[_pair_spec, _deg_spec, _b_spec, _w_spec],
    out_specs=_row_spec,
    out_shape=jax.ShapeDtypeStruct((N, D), jnp.float32),
)

_final = pl.pallas_call(
    _final_body,
    grid=(GRID,),
    in_specs=[_pair_spec, _deg_spec, _b_spec],
    out_specs=_row_spec,
    out_shape=jax.ShapeDtypeStruct((N, D), jnp.float32),
)


def kernel(features, edge_index, W1, b1, W2, b2):
    src = edge_index[0].reshape(EDGE_ROWS, B)
    dst = edge_index[1].reshape(EDGE_ROWS, B)
    b1r = b1.reshape(1, D)
    b2r = b2.reshape(1, D)

    degs = _deg_kernel(src, dst)

    h1 = _mm1(features, W1, degs)
    p1 = _agg_kernel(h1, src, dst)
    h2 = _layer2(p1, degs, b1r, W2)
    p2 = _agg_kernel(h2, src, dst)
    return _final(p2, degs, b2r)


# speedup vs baseline: 10.8632x; 1.0211x over previous
import functools

import jax
import jax.numpy as jnp
from jax import lax
from jax.experimental import pallas as pl
from jax.experimental.pallas import tpu as pltpu
from jax.experimental.pallas import tpu_sc as plsc

N = 10000
E = 320000
D = 128

NC = 2
NS = 16

B = 100
ROWS_PER_TILE = E // (NC * NS) // B
EDGE_ROWS = E // B
NBLK = 4
BLK = ROWS_PER_TILE // NBLK
N_PAD = 10240
NPT = N_PAD // NS
ZCH = 80
ZROWS = NPT // ZCH

_MESH = plsc.VectorSubcoreMesh(core_axis_name="c", subcore_axis_name="s")


def _zero_vmem(ref, nrows, ncols):
    zeros16 = jnp.zeros((16,), jnp.float32)

    def body(i, _):
        for col in range(ncols // 16):
            ref[i, pl.ds(col * 16, 16)] = zeros16
        return 0

    lax.fori_loop(0, nrows, body, 0)


def _deg_body(src_hbm, dst_hbm, out_hbm,
              deg_s, deg_d, src_v, dst_v, ones_v, zbuf, sem_a, sem_b):
    c = lax.axis_index("c")
    s = lax.axis_index("s")

    _zero_vmem(zbuf, ZROWS, 16)
    for k in range(ZCH):
        base = s * NPT + k * ZROWS
        pltpu.sync_copy(zbuf, deg_s.at[pl.ds(base, ZROWS)])
        pltpu.sync_copy(zbuf, deg_d.at[pl.ds(base, ZROWS)])

    ones16 = jnp.ones((16,), jnp.float32)

    def ones_body(i, _):
        ones_v[i, :] = ones16
        return 0

    lax.fori_loop(0, B, ones_body, 0)

    plsc.subcore_barrier()

    row0 = c * (EDGE_ROWS // NC) + s * ROWS_PER_TILE
    pltpu.sync_copy(src_hbm.at[pl.ds(row0, ROWS_PER_TILE)], src_v)
    pltpu.sync_copy(dst_hbm.at[pl.ds(row0, ROWS_PER_TILE)], dst_v)

    def chunk(j, _):
        pltpu.async_copy(ones_v, deg_s.at[src_v.at[j]], sem_a, add=True)
        pltpu.async_copy(ones_v, deg_d.at[dst_v.at[j]], sem_b, add=True)

        @pl.when(j >= 1)
        def _():
            pltpu.make_async_copy(ones_v, deg_s.at[src_v.at[j - 1]], sem_a).wait()
            pltpu.make_async_copy(ones_v, deg_d.at[dst_v.at[j - 1]], sem_b).wait()

        return 0

    lax.fori_loop(0, ROWS_PER_TILE, chunk, 0)
    last = ROWS_PER_TILE - 1
    pltpu.make_async_copy(ones_v, deg_s.at[src_v.at[last]], sem_a).wait()
    pltpu.make_async_copy(ones_v, deg_d.at[dst_v.at[last]], sem_b).wait()

    plsc.subcore_barrier()

    base = s * NPT
    pltpu.sync_copy(deg_s.at[pl.ds(base, NPT)], out_hbm.at[c, 0, pl.ds(base, NPT)])
    pltpu.sync_copy(deg_d.at[pl.ds(base, NPT)], out_hbm.at[c, 1, pl.ds(base, NPT)])


_SC_PARAMS = pltpu.CompilerParams(use_tc_tiling_on_sc=False)

_deg_kernel = pl.kernel(
    _deg_body,
    out_type=jax.ShapeDtypeStruct((NC, 2, N_PAD, 16), jnp.float32),
    mesh=_MESH,
    compiler_params=_SC_PARAMS,
    scratch_types=[
        pltpu.VMEM_SHARED((N_PAD, 16), jnp.float32),
        pltpu.VMEM_SHARED((N_PAD, 16), jnp.float32),
        pltpu.VMEM((ROWS_PER_TILE, B), jnp.int32),
        pltpu.VMEM((ROWS_PER_TILE, B), jnp.int32),
        pltpu.VMEM((B, 16), jnp.float32),
        pltpu.VMEM((ZROWS, 16), jnp.float32),
        pltpu.SemaphoreType.DMA,
        pltpu.SemaphoreType.DMA,
    ],
)


def _agg_body(h_hbm, src_hbm, dst_hbm, out_hbm,
              acc, src_v, dst_v, rows0, rows1, rows2, zbuf,
              g0, g1, g2, s0, s1, s2):
    c = lax.axis_index("c")
    s = lax.axis_index("s")

    rows = (rows0, rows1, rows2)
    gsem = (g0, g1, g2)
    ssem = (s0, s1, s2)

    _zero_vmem(zbuf, ZROWS, D)
    for k in range(ZCH):
        base = s * NPT + k * ZROWS
        pltpu.sync_copy(zbuf, acc.at[pl.ds(base, ZROWS)])

    plsc.subcore_barrier()

    row0 = c * (EDGE_ROWS // NC) + s * ROWS_PER_TILE

    for blk in range(NBLK):
        brow = row0 + blk * BLK
        pltpu.sync_copy(src_hbm.at[pl.ds(brow, BLK)], src_v)
        pltpu.sync_copy(dst_hbm.at[pl.ds(brow, BLK)], dst_v)

        pltpu.async_copy(h_hbm.at[src_v.at[0]], rows0, g0)
        pltpu.async_copy(h_hbm.at[src_v.at[1]], rows1, g1)

        def slot(j, b, bp):
            pltpu.make_async_copy(h_hbm.at[src_v.at[j]], rows[b], gsem[b]).wait()
            pltpu.async_copy(rows[b], acc.at[dst_v.at[j]], ssem[b], add=True)

            @pl.when(j >= 1)
            def _():
                pltpu.make_async_copy(rows[bp], acc.at[dst_v.at[j - 1]],
                                      ssem[bp]).wait()

            @pl.when(j + 2 < BLK)
            def _():
                pltpu.async_copy(h_hbm.at[src_v.at[j + 2]], rows[bp], gsem[bp])

        def tri(k, _):
            for i in range(3):
                slot(k * 3 + i, i, (i + 2) % 3)
            return 0

        lax.fori_loop(0, BLK // 3, tri, 0)
        for j in range(BLK - BLK % 3, BLK):
            slot(j, j % 3, (j + 2) % 3)
        bl = BLK - 1
        pltpu.make_async_copy(rows[bl % 3], acc.at[dst_v.at[bl]],
                              ssem[bl % 3]).wait()

    plsc.subcore_barrier()

    base = s * NPT
    pltpu.sync_copy(acc.at[pl.ds(base, NPT)], out_hbm.at[c, pl.ds(base, NPT)])


_agg_kernel = pl.kernel(
    _agg_body,
    out_type=jax.ShapeDtypeStruct((NC, N_PAD, D), jnp.float32),
    mesh=_MESH,
    compiler_params=_SC_PARAMS,
    scratch_types=[
        pltpu.VMEM_SHARED((N_PAD, D), jnp.float32),
        pltpu.VMEM((BLK, B), jnp.int32),
        pltpu.VMEM((BLK, B), jnp.int32),
        pltpu.VMEM((B, D), jnp.float32),
        pltpu.VMEM((B, D), jnp.float32),
        pltpu.VMEM((B, D), jnp.float32),
        pltpu.VMEM((ZROWS, D), jnp.float32),
        pltpu.SemaphoreType.DMA,
        pltpu.SemaphoreType.DMA,
        pltpu.SemaphoreType.DMA,
        pltpu.SemaphoreType.DMA,
        pltpu.SemaphoreType.DMA,
        pltpu.SemaphoreType.DMA,
    ],
)


RB = 400
GRID = N // RB


def _norms(deg_blk):
    d_src = deg_blk[0, 0] + deg_blk[1, 0]
    d_dst = deg_blk[0, 1] + deg_blk[1, 1]
    n_out = lax.rsqrt(jnp.clip(d_src[:, 0:1], 1.0, None))
    n_in = lax.rsqrt(jnp.clip(d_dst[:, 0:1], 1.0, None))
    return n_out, n_in


def _mm1_body(x_ref, w_ref, deg_ref, o_ref):
    n_out, _ = _norms(deg_ref[...])
    o_ref[...] = jnp.dot(x_ref[...], w_ref[...],
                         preferred_element_type=jnp.float32) * n_out


def _layer2_body(p_ref, deg_ref, b1_ref, w2_ref, o_ref):
    n_out, n_in = _norms(deg_ref[...])
    h = jnp.maximum((p_ref[0] + p_ref[1]) * n_in + b1_ref[...], 0.0)
    o_ref[...] = jnp.dot(h, w2_ref[...],
                         preferred_element_type=jnp.float32) * n_out


def _final_body(p_ref, deg_ref, b2_ref, o_ref):
    _, n_in = _norms(deg_ref[...])
    o_ref[...] = (p_ref[0] + p_ref[1]) * n_in + b2_ref[...]


_deg_spec = pl.BlockSpec((2, 2, RB, 16), lambda i: (0, 0, i, 0))
_row_spec = pl.BlockSpec((RB, D), lambda i: (i, 0))
_pair_spec = pl.BlockSpec((2, RB, D), lambda i: (0, i, 0))
_w_spec = pl.BlockSpec((D, D), lambda i: (0, 0))
_b_spec = pl.BlockSpec((1, D), lambda i: (0, 0))

_mm1 = pl.pallas_call(
    _mm1_body,
    grid=(GRID,),
    in_specs=[_row_spec, _w_spec, _deg_spec],
    out_specs=_row_spec,
    out_shape=jax.ShapeDtypeStruct((N, D), jnp.float32),
)

_layer2 = pl.pallas_call(
    _layer2_body,
    grid=(GRID,),
    in_specs=[_pair_spec, _deg_spec, _b_spec, _w_spec],
    out_specs=_row_spec,
    out_shape=jax.ShapeDtypeStruct((N, D), jnp.float32),
)

_final = pl.pallas_call(
    _final_body,
    grid=(GRID,),
    in_specs=[_pair_spec, _deg_spec, _b_spec],
    out_specs=_row_spec,
    out_shape=jax.ShapeDtypeStruct((N, D), jnp.float32),
)


def kernel(features, edge_index, W1, b1, W2, b2):
    src = edge_index[0].reshape(EDGE_ROWS, B)
    dst = edge_index[1].reshape(EDGE_ROWS, B)
    b1r = b1.reshape(1, D)
    b2r = b2.reshape(1, D)

    degs = _deg_kernel(src, dst)

    h1 = _mm1(features, W1, degs)
    p1 = _agg_kernel(h1, src, dst)
    h2 = _layer2(p1, degs, b1r, W2)
    p2 = _agg_kernel(h2, src, dst)
    return _final(p2, degs, b2r)
